# baseline (speedup 1.0000x reference)
import functools

import jax
import jax.numpy as jnp
from jax import lax
from jax.experimental import pallas as pl
from jax.experimental.pallas import tpu as pltpu
from jax.experimental.pallas import tpu_sc as plsc

NC = 2
NS = 16
NW = NC * NS
BATCH = 16384
HIDDEN = 64
CHUNK = 128
B_PER_W = BATCH // NW
NCHUNK = B_PER_W // CHUNK

NTAB = 1000000
RSB = 131072
RSGRID = (NTAB + RSB - 1) // RSB
RS_LEN = RSGRID * RSB


def _rowsum_body(in_ref, out_ref):
    @pl.when(pl.program_id(1) == 0)
    def _():
        out_ref[...] = jnp.zeros_like(out_ref)

    out_ref[...] += jnp.sum(in_ref[...], axis=0)


def _rowsum(tT):
    return pl.pallas_call(
        _rowsum_body,
        grid=(RSGRID, HIDDEN // 8),
        in_specs=[pl.BlockSpec((8, RSB), lambda j, r: (r, j))],
        out_specs=pl.BlockSpec((RSB,), lambda j, r: (j,)),
        out_shape=jax.ShapeDtypeStruct((RS_LEN,), jnp.float32),
    )(tT)


def _sc_body(users_ref, items_ref, uf_hbm, if_hbm, rsu_hbm, rsi_hbm, out_hbm,
             uidx_v, iidx_v, uf_v, if_v, rsu_v, rsi_v, out_v, sem):
    wid = lax.axis_index("s") * NC + lax.axis_index("c")
    base = wid * B_PER_W
    row0 = wid * NCHUNK

    pltpu.sync_copy(users_ref.at[pl.ds(row0, NCHUNK)], uidx_v)
    pltpu.sync_copy(items_ref.at[pl.ds(row0, NCHUNK)], iidx_v)

    lanes = jax.lax.iota(jnp.int32, 16)
    last_lane = lanes == 15

    for c in range(NCHUNK):
        cp0 = pltpu.async_copy(uf_hbm.at[uidx_v.at[c]], uf_v, sem)
        cp1 = pltpu.async_copy(if_hbm.at[iidx_v.at[c]], if_v, sem)
        cp2 = pltpu.async_copy(rsu_hbm.at[uidx_v.at[c]], rsu_v.at[c], sem)
        cp3 = pltpu.async_copy(rsi_hbm.at[iidx_v.at[c]], rsi_v.at[c], sem)
        cp0.wait()
        cp1.wait()
        cp2.wait()
        cp3.wait()

        def group(g, _):
            for l in range(16):
                e = g * 16 + l
                acc = None
                for j in range(HIDDEN // 16):
                    sj = pl.ds(j * 16, 16)
                    t = uf_v[e, sj] * if_v[e, sj]
                    acc = t if acc is None else acc + t
                sums = plsc.cumsum(acc)
                plsc.store_scatter(out_v,
                                   [jnp.full((16,), c * CHUNK + e, jnp.int32)],
                                   sums, mask=last_lane)
            return 0

        lax.fori_loop(0, CHUNK // 16, group, 0)

        def bias(g, _):
            so = pl.ds(c * CHUNK + g * 16, 16)
            sg = pl.ds(g * 16, 16)
            out_v[so] = out_v[so] + rsu_v[c, sg] + rsi_v[c, sg]
            return 0

        lax.fori_loop(0, CHUNK // 16, bias, 0)

    pltpu.sync_copy(out_v, out_hbm.at[pl.ds(base, B_PER_W)])


@functools.partial(jax.jit, static_argnames=())
def kernel(users, items, user_factors, item_factors, user_biases, item_biases):
    rsu = _rowsum(user_biases.T)
    rsi = _rowsum(item_biases.T)

    mesh = plsc.VectorSubcoreMesh(
        core_axis_name="c", subcore_axis_name="s",
        num_cores=NC, num_subcores=NS)
    f = pl.kernel(
        _sc_body,
        out_type=jax.ShapeDtypeStruct((BATCH,), jnp.float32),
        mesh=mesh,
        compiler_params=pltpu.CompilerParams(needs_layout_passes=False,
                                             use_tc_tiling_on_sc=False),
        scratch_types=[
            pltpu.VMEM((NCHUNK, CHUNK), jnp.int32),
            pltpu.VMEM((NCHUNK, CHUNK), jnp.int32),
            pltpu.VMEM((CHUNK, HIDDEN), jnp.float32),
            pltpu.VMEM((CHUNK, HIDDEN), jnp.float32),
            pltpu.VMEM((NCHUNK, CHUNK), jnp.float32),
            pltpu.VMEM((NCHUNK, CHUNK), jnp.float32),
            pltpu.VMEM((B_PER_W,), jnp.float32),
            pltpu.SemaphoreType.DMA,
        ],
    )
    out = f(users.reshape(BATCH // CHUNK, CHUNK),
            items.reshape(BATCH // CHUNK, CHUNK),
            user_factors, item_factors, rsu, rsi)
    return out.reshape(BATCH, 1)


# speedup vs baseline: 1.0120x; 1.0120x over previous
import functools

import jax
import jax.numpy as jnp
from jax import lax
from jax.experimental import pallas as pl
from jax.experimental.pallas import tpu as pltpu
from jax.experimental.pallas import tpu_sc as plsc

NC = 2
NS = 16
NW = NC * NS
BATCH = 16384
HIDDEN = 64
CHUNK = 128
B_PER_W = BATCH // NW
NCHUNK = B_PER_W // CHUNK

NTAB = 1000000
RSB = 131072
RSGRID = (NTAB + RSB - 1) // RSB
RS_LEN = RSGRID * RSB


def _rowsum_body(inu_ref, ini_ref, outu_ref, outi_ref):
    @pl.when(pl.program_id(1) == 0)
    def _():
        outu_ref[...] = jnp.zeros_like(outu_ref)
        outi_ref[...] = jnp.zeros_like(outi_ref)

    outu_ref[...] += jnp.sum(inu_ref[...], axis=0)
    outi_ref[...] += jnp.sum(ini_ref[...], axis=0)


def _rowsum(tTu, tTi):
    bs = pl.BlockSpec((8, RSB), lambda j, r: (r, j))
    os = pl.BlockSpec((RSB,), lambda j, r: (j,))
    return pl.pallas_call(
        _rowsum_body,
        grid=(RSGRID, HIDDEN // 8),
        in_specs=[bs, bs],
        out_specs=[os, os],
        out_shape=[jax.ShapeDtypeStruct((RS_LEN,), jnp.float32),
                   jax.ShapeDtypeStruct((RS_LEN,), jnp.float32)],
    )(tTu, tTi)


def _sc_body(users_ref, items_ref, uf_hbm, if_hbm, rsu_hbm, rsi_hbm, out_hbm,
             uidx_v, iidx_v, uf_v, if_v, rsu_v, rsi_v, out_v, sem):
    wid = lax.axis_index("s") * NC + lax.axis_index("c")
    base = wid * B_PER_W
    row0 = wid * NCHUNK

    pltpu.sync_copy(users_ref.at[pl.ds(row0, NCHUNK)], uidx_v)
    pltpu.sync_copy(items_ref.at[pl.ds(row0, NCHUNK)], iidx_v)

    lanes = jax.lax.iota(jnp.int32, 16)
    last_lane = lanes == 15

    for c in range(NCHUNK):
        cp0 = pltpu.async_copy(uf_hbm.at[uidx_v.at[c]], uf_v, sem)
        cp1 = pltpu.async_copy(if_hbm.at[iidx_v.at[c]], if_v, sem)
        cp2 = pltpu.async_copy(rsu_hbm.at[uidx_v.at[c]], rsu_v.at[c], sem)
        cp3 = pltpu.async_copy(rsi_hbm.at[iidx_v.at[c]], rsi_v.at[c], sem)
        cp0.wait()
        cp1.wait()
        cp2.wait()
        cp3.wait()

        def group(g, _):
            for l in range(16):
                e = g * 16 + l
                acc = None
                for j in range(HIDDEN // 16):
                    sj = pl.ds(j * 16, 16)
                    t = uf_v[e, sj] * if_v[e, sj]
                    acc = t if acc is None else acc + t
                sums = plsc.cumsum(acc)
                plsc.store_scatter(out_v,
                                   [jnp.full((16,), c * CHUNK + e, jnp.int32)],
                                   sums, mask=last_lane)
            return 0

        lax.fori_loop(0, CHUNK // 16, group, 0)

        def bias(g, _):
            so = pl.ds(c * CHUNK + g * 16, 16)
            sg = pl.ds(g * 16, 16)
            out_v[so] = out_v[so] + rsu_v[c, sg] + rsi_v[c, sg]
            return 0

        lax.fori_loop(0, CHUNK // 16, bias, 0)

    pltpu.sync_copy(out_v, out_hbm.at[pl.ds(base, B_PER_W)])


@functools.partial(jax.jit, static_argnames=())
def kernel(users, items, user_factors, item_factors, user_biases, item_biases):
    rsu, rsi = _rowsum(user_biases.T, item_biases.T)

    mesh = plsc.VectorSubcoreMesh(
        core_axis_name="c", subcore_axis_name="s",
        num_cores=NC, num_subcores=NS)
    f = pl.kernel(
        _sc_body,
        out_type=jax.ShapeDtypeStruct((BATCH,), jnp.float32),
        mesh=mesh,
        compiler_params=pltpu.CompilerParams(needs_layout_passes=False,
                                             use_tc_tiling_on_sc=False),
        scratch_types=[
            pltpu.VMEM((NCHUNK, CHUNK), jnp.int32),
            pltpu.VMEM((NCHUNK, CHUNK), jnp.int32),
            pltpu.VMEM((CHUNK, HIDDEN), jnp.float32),
            pltpu.VMEM((CHUNK, HIDDEN), jnp.float32),
            pltpu.VMEM((NCHUNK, CHUNK), jnp.float32),
            pltpu.VMEM((NCHUNK, CHUNK), jnp.float32),
            pltpu.VMEM((B_PER_W,), jnp.float32),
            pltpu.SemaphoreType.DMA,
        ],
    )
    out = f(users.reshape(BATCH // CHUNK, CHUNK),
            items.reshape(BATCH // CHUNK, CHUNK),
            user_factors, item_factors, rsu, rsi)
    return out.reshape(BATCH, 1)
